# baseline (speedup 1.0000x reference)
import jax
import jax.numpy as jnp
from jax import lax
from jax.experimental import pallas as pl
from jax.experimental.pallas import tpu as pltpu
from jax.experimental.pallas import tpu_sc as plsc

M = 262144
D = 128
B = 65536

NC = 2
NS = 16
NW = NC * NS
ROWS_PER_CHUNK = 128
B_PER_W = B // NW
CHUNKS_PER_W = B_PER_W // ROWS_PER_CHUNK


def _copy_body(h_hbm, c_hbm, ho_hbm, co_hbm, sem_h, sem_c):
    ch = pltpu.make_async_copy(h_hbm, ho_hbm, sem_h)
    cc = pltpu.make_async_copy(c_hbm, co_hbm, sem_c)
    ch.start()
    cc.start()
    ch.wait()
    cc.wait()


_tc_copy = pl.pallas_call(
    _copy_body,
    in_specs=[pl.BlockSpec(memory_space=pl.ANY)] * 2,
    out_specs=[pl.BlockSpec(memory_space=pl.ANY)] * 2,
    out_shape=(
        jax.ShapeDtypeStruct((M, D), jnp.float32),
        jax.ShapeDtypeStruct((M, D), jnp.float32),
    ),
    scratch_shapes=[pltpu.SemaphoreType.DMA] * 2,
)


def _sc_scatter_body(idx_hbm, val_hbm, h_ref, c_ref, idx_v, val_v, sem_i, sem_h, sem_c):
    wid = lax.axis_index("s") * NC + lax.axis_index("c")
    pltpu.async_copy(idx_hbm.at[pl.ds(wid * CHUNKS_PER_W, CHUNKS_PER_W)], idx_v, sem_i).wait()

    def chunk(j, _):
        row0 = wid * B_PER_W + j * ROWS_PER_CHUNK
        pltpu.async_copy(val_hbm.at[pl.ds(row0, ROWS_PER_CHUNK)], val_v, sem_i).wait()
        idx_row = idx_v.at[j]
        cp_h = pltpu.make_async_copy(val_v, h_ref.at[idx_row], sem_h)
        cp_c = pltpu.make_async_copy(val_v, c_ref.at[idx_row], sem_c)
        cp_h.start()
        cp_c.start()
        cp_h.wait()
        cp_c.wait()
        return ()

    lax.fori_loop(0, CHUNKS_PER_W, chunk, ())


_sc_scatter = pl.kernel(
    _sc_scatter_body,
    out_type=(),
    mesh=plsc.VectorSubcoreMesh(core_axis_name="c", subcore_axis_name="s",
                                num_cores=NC, num_subcores=NS),
    scratch_types=[
        pltpu.VMEM((CHUNKS_PER_W, ROWS_PER_CHUNK), jnp.int32),
        pltpu.VMEM((ROWS_PER_CHUNK, D), jnp.float32),
        pltpu.SemaphoreType.DMA,
        pltpu.SemaphoreType.DMA,
        pltpu.SemaphoreType.DMA,
    ],
)


def kernel(h, c, idx, val):
    idx2d = idx.astype(jnp.int32).reshape(B // ROWS_PER_CHUNK, ROWS_PER_CHUNK)
    h_new, c_new = _tc_copy(h, c)
    h_ref = jax.new_ref(h_new)
    c_ref = jax.new_ref(c_new)
    _sc_scatter(idx2d, val, h_ref, c_ref)
    return (h_ref[...], c_ref[...])
